# Initial kernel scaffold; baseline (speedup 1.0000x reference)
import jax
import jax.numpy as jnp
from jax.experimental import pallas as pl


def kernel(x, node_idx, hedge_idx, W1, b1, W2, b2):
    raise NotImplementedError("write your pallas kernel here")


# speedup vs baseline: 3.1418x; 3.1418x over previous
import jax
import jax.numpy as jnp
from jax import lax
from jax.experimental import pallas as pl
from jax.experimental.pallas import tpu as pltpu
from jax.experimental.pallas import tpu_sc as plsc

NV = 10000
NE = 5000
NP = 320000
D1 = 128
D2 = 64

NC = 2
NS = 16
NW = NC * NS

NVp = 10240
NEp = 5120
SH = NVp // NC
DUMP = SH

C = 80
PW = NP // NW
NCHUNK = PW // C
PWN = NP // NS
NCHUNK_N = PWN // C

_mesh = plsc.VectorSubcoreMesh(core_axis_name="c", subcore_axis_name="s")


def _sweep_full_body(table, gidx, sidx, zrow, z1, ones, pE, cntE,
                     idxg, idxs, rows, zbuf, z1v, ones_v, accE, dE, sem):
    cid = lax.axis_index("c")
    sid = lax.axis_index("s")
    wid = sid * NC + cid
    rpsE = NEp // NS

    pltpu.sync_copy(zrow, zbuf)
    pltpu.sync_copy(z1, z1v)
    pltpu.sync_copy(ones, ones_v)
    pltpu.sync_copy(zbuf, accE.at[pl.ds(sid * rpsE, rpsE)])
    pltpu.sync_copy(z1v, dE.at[pl.ds(sid * rpsE, rpsE)])
    plsc.subcore_barrier()

    @pl.loop(0, NCHUNK)
    def _chunk(i):
        off = pl.multiple_of(wid * PW + i * C, 8)
        pltpu.sync_copy(gidx.at[pl.ds(off, C)], idxg)
        pltpu.async_copy(table.at[idxg], rows, sem).wait()
        pltpu.sync_copy(sidx.at[pl.ds(off, C)], idxs)
        pltpu.sync_copy(rows, accE.at[idxs], add=True)
        pltpu.sync_copy(ones_v, dE.at[idxs], add=True)

    plsc.subcore_barrier()
    pltpu.sync_copy(accE.at[pl.ds(sid * rpsE, rpsE)],
                    pE.at[cid, pl.ds(sid * rpsE, rpsE)])
    pltpu.sync_copy(dE.at[pl.ds(sid * rpsE, rpsE)], z1v)
    pltpu.sync_copy(z1v, cntE.at[pl.ds(cid * NEp + sid * rpsE, rpsE)])


_SWEEP_FULL = pl.kernel(
    _sweep_full_body,
    out_type=(
        jax.ShapeDtypeStruct((NC, NEp, D1), jnp.float32),
        jax.ShapeDtypeStruct((NC * NEp,), jnp.float32),
    ),
    mesh=_mesh,
    scratch_types=[
        pltpu.VMEM((C,), jnp.int32),
        pltpu.VMEM((C,), jnp.int32),
        pltpu.VMEM((C, D1), jnp.float32),
        pltpu.VMEM((NEp // NS, D1), jnp.float32),
        pltpu.VMEM((NEp // NS,), jnp.float32),
        pltpu.VMEM((C,), jnp.float32),
        pltpu.VMEM_SHARED((NEp, D1), jnp.float32),
        pltpu.VMEM_SHARED((NEp,), jnp.float32),
        pltpu.SemaphoreType.DMA,
    ],
)


def _sweep_half_body(table, gidx, sidx, zrow, z1, ones, out, cntV,
                     idxg, idxs, idxr, rows, zbuf, z1v, ones_v, acc, dV, sem):
    cid = lax.axis_index("c")
    sid = lax.axis_index("s")
    rps = SH // NS
    lo = cid * SH
    hi = lo + SH

    pltpu.sync_copy(zrow, zbuf)
    pltpu.sync_copy(z1, z1v)
    pltpu.sync_copy(ones, ones_v)
    pltpu.sync_copy(z1v, dV.at[pl.ds(sid * rps, rps)])
    pltpu.sync_copy(zbuf, acc.at[pl.ds(sid * rps, rps)])
    plsc.subcore_barrier()

    @pl.loop(0, NCHUNK_N)
    def _chunk(i):
        off = pl.multiple_of(sid * PWN + i * C, 8)
        pltpu.sync_copy(gidx.at[pl.ds(off, C)], idxg)
        pltpu.async_copy(table.at[idxg], rows, sem).wait()
        pltpu.sync_copy(sidx.at[pl.ds(off, C)], idxs)
        for j in range(C // 16):
            sv = idxs[pl.ds(j * 16, 16)]
            m = jnp.logical_and(sv >= lo, sv < hi)
            idxr[pl.ds(j * 16, 16)] = jnp.where(m, sv - lo, DUMP)
        pltpu.sync_copy(rows, acc.at[idxr], add=True)
        pltpu.sync_copy(ones_v, dV.at[idxr], add=True)

    plsc.subcore_barrier()
    pltpu.sync_copy(acc.at[pl.ds(sid * rps, rps)],
                    out.at[pl.ds(cid * SH + sid * rps, rps)])
    pltpu.sync_copy(dV.at[pl.ds(sid * rps, rps)], z1v.at[pl.ds(0, rps)])
    pltpu.sync_copy(z1v.at[pl.ds(0, rps)],
                    cntV.at[pl.ds(cid * SH + sid * rps, rps)])


_SWEEP_HALF = pl.kernel(
    _sweep_half_body,
    out_type=(
        jax.ShapeDtypeStruct((NVp, D1), jnp.float32),
        jax.ShapeDtypeStruct((NVp,), jnp.float32),
    ),
    mesh=_mesh,
    scratch_types=[
        pltpu.VMEM((C,), jnp.int32),
        pltpu.VMEM((C,), jnp.int32),
        pltpu.VMEM((C,), jnp.int32),
        pltpu.VMEM((C, D1), jnp.float32),
        pltpu.VMEM((SH // NS, D1), jnp.float32),
        pltpu.VMEM((SH // NS,), jnp.float32),
        pltpu.VMEM((C,), jnp.float32),
        pltpu.VMEM_SHARED((SH + 16, D1), jnp.float32),
        pltpu.VMEM_SHARED((SH + 16,), jnp.float32),
        pltpu.SemaphoreType.DMA,
    ],
)


def _linear_body(x_ref, w_ref, b_ref, o_ref):
    o_ref[...] = (
        jnp.dot(x_ref[...], w_ref[...], preferred_element_type=jnp.float32)
        + b_ref[...]
    )


def _tc_linear(x, W, b, br):
    n, d_in = x.shape
    d_out = W.shape[1]
    return pl.pallas_call(
        _linear_body,
        grid=(n // br,),
        in_specs=[
            pl.BlockSpec((br, d_in), lambda i: (i, 0)),
            pl.BlockSpec((d_in, d_out), lambda i: (0, 0)),
            pl.BlockSpec((1, d_out), lambda i: (0, 0)),
        ],
        out_specs=pl.BlockSpec((br, d_out), lambda i: (i, 0)),
        out_shape=jax.ShapeDtypeStruct((n, d_out), jnp.float32),
    )(x, W, b)


def _combine_body(p_ref, c_ref, o_ref):
    p = p_ref[...]
    deg = jnp.maximum(c_ref[0] + c_ref[1], 1.0)
    o_ref[...] = (p[0] + p[1]) / deg


def _tc_combine(p, cnt3, br):
    _, s, d = p.shape
    return pl.pallas_call(
        _combine_body,
        grid=(s // br,),
        in_specs=[
            pl.BlockSpec((NC, br, d), lambda i: (0, i, 0)),
            pl.BlockSpec((NC, br, 1), lambda i: (0, i, 0)),
        ],
        out_specs=pl.BlockSpec((br, d), lambda i: (i, 0)),
        out_shape=jax.ShapeDtypeStruct((s, d), jnp.float32),
    )(p, cnt3)


def _relu_norm_body(p_ref, c_ref, o_ref):
    deg = jnp.maximum(c_ref[...], 1.0)
    o_ref[...] = jnp.maximum(p_ref[...] / deg, 0.0)


def _tc_relu_norm(p, cnt2, br):
    s, d = p.shape
    return pl.pallas_call(
        _relu_norm_body,
        grid=(s // br,),
        in_specs=[
            pl.BlockSpec((br, d), lambda i: (i, 0)),
            pl.BlockSpec((br, 1), lambda i: (i, 0)),
        ],
        out_specs=pl.BlockSpec((br, d), lambda i: (i, 0)),
        out_shape=jax.ShapeDtypeStruct((s, d), jnp.float32),
    )(p, cnt2)


def _final_body(g_ref, c_ref, w_ref, b_ref, o_ref):
    cnt = c_ref[...]
    deg = jnp.maximum(cnt, 1.0)
    z = (jnp.dot(g_ref[...], w_ref[...], preferred_element_type=jnp.float32)
         + cnt * b_ref[...]) / deg
    m = jnp.max(z, axis=-1, keepdims=True)
    e = jnp.exp(z - m)
    o_ref[...] = e / jnp.sum(e, axis=-1, keepdims=True)


def _tc_final(g, cnt2, W, b, br):
    s, d_in = g.shape
    d_out = W.shape[1]
    return pl.pallas_call(
        _final_body,
        grid=(s // br,),
        in_specs=[
            pl.BlockSpec((br, d_in), lambda i: (i, 0)),
            pl.BlockSpec((br, 1), lambda i: (i, 0)),
            pl.BlockSpec((d_in, d_out), lambda i: (0, 0)),
            pl.BlockSpec((1, d_out), lambda i: (0, 0)),
        ],
        out_specs=pl.BlockSpec((br, d_out), lambda i: (i, 0)),
        out_shape=jax.ShapeDtypeStruct((s, d_out), jnp.float32),
    )(g, cnt2, W, b)


def kernel(x, node_idx, hedge_idx, W1, b1, W2, b2):
    nidx = node_idx.astype(jnp.int32)
    hidx = hedge_idx.astype(jnp.int32)
    zrow = jnp.zeros((NEp // NS, D1), jnp.float32)
    z1 = jnp.zeros((NEp // NS,), jnp.float32)
    ones = jnp.ones((C,), jnp.float32)

    xp = jnp.concatenate([x, jnp.zeros((NVp - NV, D1), jnp.float32)], axis=0)
    xt1 = _tc_linear(xp, W1, b1.reshape(1, -1), br=512)

    pA, cntE = _SWEEP_FULL(xt1, nidx, hidx, zrow, z1, ones)
    cntE3 = cntE.reshape(NC, NEp, 1)

    e1 = _tc_combine(pA, cntE3, br=256)
    hB, cntV = _SWEEP_HALF(e1, hidx, nidx, zrow, z1, ones)
    cntV2 = cntV.reshape(NVp, 1)
    h = _tc_relu_norm(hB, cntV2, br=512)
    pC, _ = _SWEEP_FULL(h, nidx, hidx, zrow, z1, ones)
    f1 = _tc_combine(pC, cntE3, br=256)
    G, _ = _SWEEP_HALF(f1, hidx, nidx, zrow, z1, ones)
    out = _tc_final(G, cntV2, W2, b2.reshape(1, -1), br=512)
    return out[:NV]


# speedup vs baseline: 6.9735x; 2.2195x over previous
import jax
import jax.numpy as jnp
from jax import lax
from jax.experimental import pallas as pl
from jax.experimental.pallas import tpu as pltpu
from jax.experimental.pallas import tpu_sc as plsc

NV = 10000
NE = 5000
NP = 320000
D1 = 128
D2 = 64

NC = 2
NS = 16
NW = NC * NS

NVp = 10048
NEp = 5024
SH = NVp // NC
DUMP = SH

C = 80
PW = NP // NW
NCHUNK = PW // C
PWN = NP // NS
NCHUNK_N = PWN // C

_mesh = plsc.VectorSubcoreMesh(core_axis_name="c", subcore_axis_name="s")


RING = 4
ZR = 32
NZ = NEp // ZR
NZT = (NZ + NS - 1) // NS


def _run_pipelined(table, gidx_hbm, sidx_hbm, base, nchunks, remap,
                   idxg_f, idxs_f, idxgc, idxrc, rows, ones_v, acc, dacc,
                   gsem, scsem, sosem):
    pltpu.sync_copy(gidx_hbm.at[pl.ds(base, nchunks * C)], idxg_f)
    pltpu.sync_copy(sidx_hbm.at[pl.ds(base, nchunks * C)], idxs_f)

    def move_g(c, s):
        for j in range(C // 16):
            idxgc[s, pl.ds(j * 16, 16)] = idxg_f[pl.ds(c * C + j * 16, 16)]

    def move_s(c, s):
        for j in range(C // 16):
            idxrc[s, pl.ds(j * 16, 16)] = remap(
                idxs_f[pl.ds(c * C + j * 16, 16)])

    def issue_gather(c, s):
        move_g(c, s)
        pltpu.async_copy(table.at[idxgc.at[s]], rows.at[s], gsem.at[s])

    def wait_g(s):
        pltpu.make_async_copy(table.at[idxgc.at[s]], rows.at[s],
                              gsem.at[s]).wait()

    def issue_scatters(c, s):
        move_s(c, s)
        pltpu.async_copy(rows.at[s], acc.at[idxrc.at[s]], scsem.at[s],
                         add=True)
        pltpu.async_copy(ones_v, dacc.at[idxrc.at[s]], sosem.at[s],
                         add=True)

    def wait_sc(s):
        pltpu.make_async_copy(rows.at[s], acc.at[idxrc.at[s]],
                              scsem.at[s]).wait()
        pltpu.make_async_copy(ones_v, dacc.at[idxrc.at[s]],
                              sosem.at[s]).wait()

    for s in range(RING):
        issue_gather(s, s)

    K = (nchunks - RING) // RING
    E = nchunks - RING * K - RING

    @pl.loop(0, K)
    def _grp(k):
        for s in range(RING):
            wait_g(s)
            issue_scatters(RING * k + s, s)
        for s in range(RING):
            wait_sc(s)
            issue_gather(RING * k + RING + s, s)

    for s in range(RING):
        wait_g(s)
        issue_scatters(RING * K + s, s)
    for t in range(E):
        wait_sc(t)
        issue_gather(RING * K + RING + t, t)
        wait_g(t)
        issue_scatters(RING * K + RING + t, t)
    for s in range(RING):
        wait_sc(s)


def _sweep_full_body(table, gidx, sidx, zrow, z1, ones, pE, cntE,
                     idxg_f, idxs_f, idxgc, idxrc, rows, zbuf, z1v, ones_v,
                     accE, dE, gsem, scsem, sosem):
    cid = lax.axis_index("c")
    sid = lax.axis_index("s")
    wid = sid * NC + cid

    pltpu.sync_copy(zrow, zbuf)
    pltpu.sync_copy(z1, z1v)
    pltpu.sync_copy(ones, ones_v)
    for t in range(NZT):
        zi = sid + t * NS

        @pl.when(zi < NZ)
        def _zero():
            pltpu.sync_copy(zbuf, accE.at[pl.ds(zi * ZR, ZR)])
            pltpu.sync_copy(z1v, dE.at[pl.ds(zi * ZR, ZR)])

    plsc.subcore_barrier()

    _run_pipelined(table, gidx, sidx, wid * PW, NCHUNK, lambda v: v,
                   idxg_f, idxs_f, idxgc, idxrc, rows, ones_v, accE, dE,
                   gsem, scsem, sosem)

    plsc.subcore_barrier()
    for t in range(NZT):
        zi = sid + t * NS

        @pl.when(zi < NZ)
        def _wout():
            pltpu.sync_copy(accE.at[pl.ds(zi * ZR, ZR)],
                            pE.at[cid, pl.ds(zi * ZR, ZR)])
            pltpu.sync_copy(dE.at[pl.ds(zi * ZR, ZR)], z1v)
            pltpu.sync_copy(z1v, cntE.at[pl.ds(cid * NEp + zi * ZR, ZR)])


_SWEEP_FULL = pl.kernel(
    _sweep_full_body,
    out_type=(
        jax.ShapeDtypeStruct((NC, NEp, D1), jnp.float32),
        jax.ShapeDtypeStruct((NC * NEp,), jnp.float32),
    ),
    mesh=_mesh,
    scratch_types=[
        pltpu.VMEM((NCHUNK * C,), jnp.int32),
        pltpu.VMEM((NCHUNK * C,), jnp.int32),
        pltpu.VMEM((RING, C), jnp.int32),
        pltpu.VMEM((RING, C), jnp.int32),
        pltpu.VMEM((RING, C, D1), jnp.float32),
        pltpu.VMEM((ZR, D1), jnp.float32),
        pltpu.VMEM((ZR,), jnp.float32),
        pltpu.VMEM((C,), jnp.float32),
        pltpu.VMEM_SHARED((NEp, D1), jnp.float32),
        pltpu.VMEM_SHARED((NEp,), jnp.float32),
        pltpu.SemaphoreType.DMA((RING,)),
        pltpu.SemaphoreType.DMA((RING,)),
        pltpu.SemaphoreType.DMA((RING,)),
    ],
)


def _sweep_half_body(table, gidx, sidx, zrow, z1, ones, out, cntV,
                     idxg_f, idxs_f, idxgc, idxrc, rows, zbuf, z1v, ones_v,
                     acc, dV, gsem, scsem, sosem):
    cid = lax.axis_index("c")
    sid = lax.axis_index("s")
    lo = cid * SH
    hi = lo + SH

    pltpu.sync_copy(zrow, zbuf)
    pltpu.sync_copy(z1, z1v)
    pltpu.sync_copy(ones, ones_v)
    for t in range(NZT):
        zi = sid + t * NS

        @pl.when(zi < NZ)
        def _zero():
            pltpu.sync_copy(zbuf, acc.at[pl.ds(zi * ZR, ZR)])
            pltpu.sync_copy(z1v, dV.at[pl.ds(zi * ZR, ZR)])

    plsc.subcore_barrier()

    def remap(sv):
        m = jnp.logical_and(sv >= lo, sv < hi)
        return jnp.where(m, sv - lo, DUMP)

    _run_pipelined(table, gidx, sidx, sid * PWN, NCHUNK_N, remap,
                   idxg_f, idxs_f, idxgc, idxrc, rows, ones_v, acc, dV,
                   gsem, scsem, sosem)

    plsc.subcore_barrier()
    for t in range(NZT):
        zi = sid + t * NS

        @pl.when(zi < NZ)
        def _wout():
            pltpu.sync_copy(acc.at[pl.ds(zi * ZR, ZR)],
                            out.at[pl.ds(cid * SH + zi * ZR, ZR)])
            pltpu.sync_copy(dV.at[pl.ds(zi * ZR, ZR)], z1v)
            pltpu.sync_copy(z1v, cntV.at[pl.ds(cid * SH + zi * ZR, ZR)])


_SWEEP_HALF = pl.kernel(
    _sweep_half_body,
    out_type=(
        jax.ShapeDtypeStruct((NVp, D1), jnp.float32),
        jax.ShapeDtypeStruct((NVp,), jnp.float32),
    ),
    mesh=_mesh,
    scratch_types=[
        pltpu.VMEM((NCHUNK_N * C,), jnp.int32),
        pltpu.VMEM((NCHUNK_N * C,), jnp.int32),
        pltpu.VMEM((RING, C), jnp.int32),
        pltpu.VMEM((RING, C), jnp.int32),
        pltpu.VMEM((RING, C, D1), jnp.float32),
        pltpu.VMEM((ZR, D1), jnp.float32),
        pltpu.VMEM((ZR,), jnp.float32),
        pltpu.VMEM((C,), jnp.float32),
        pltpu.VMEM_SHARED((SH + 8, D1), jnp.float32),
        pltpu.VMEM_SHARED((SH + 8,), jnp.float32),
        pltpu.SemaphoreType.DMA((RING,)),
        pltpu.SemaphoreType.DMA((RING,)),
        pltpu.SemaphoreType.DMA((RING,)),
    ],
)


def _linear_body(x_ref, w_ref, b_ref, o_ref):
    o_ref[...] = (
        jnp.dot(x_ref[...], w_ref[...], preferred_element_type=jnp.float32)
        + b_ref[...]
    )


def _tc_linear(x, W, b, br):
    n, d_in = x.shape
    d_out = W.shape[1]
    return pl.pallas_call(
        _linear_body,
        grid=(n // br,),
        in_specs=[
            pl.BlockSpec((br, d_in), lambda i: (i, 0)),
            pl.BlockSpec((d_in, d_out), lambda i: (0, 0)),
            pl.BlockSpec((1, d_out), lambda i: (0, 0)),
        ],
        out_specs=pl.BlockSpec((br, d_out), lambda i: (i, 0)),
        out_shape=jax.ShapeDtypeStruct((n, d_out), jnp.float32),
    )(x, W, b)


def _combine_body(p_ref, c_ref, o_ref):
    p = p_ref[...]
    deg = jnp.maximum(c_ref[0] + c_ref[1], 1.0)
    o_ref[...] = (p[0] + p[1]) / deg


def _tc_combine(p, cnt3, br):
    _, s, d = p.shape
    return pl.pallas_call(
        _combine_body,
        grid=(s // br,),
        in_specs=[
            pl.BlockSpec((NC, br, d), lambda i: (0, i, 0)),
            pl.BlockSpec((NC, br, 1), lambda i: (0, i, 0)),
        ],
        out_specs=pl.BlockSpec((br, d), lambda i: (i, 0)),
        out_shape=jax.ShapeDtypeStruct((s, d), jnp.float32),
    )(p, cnt3)


def _relu_norm_body(p_ref, c_ref, o_ref):
    deg = jnp.maximum(c_ref[...], 1.0)
    o_ref[...] = jnp.maximum(p_ref[...] / deg, 0.0)


def _tc_relu_norm(p, cnt2, br):
    s, d = p.shape
    return pl.pallas_call(
        _relu_norm_body,
        grid=(s // br,),
        in_specs=[
            pl.BlockSpec((br, d), lambda i: (i, 0)),
            pl.BlockSpec((br, 1), lambda i: (i, 0)),
        ],
        out_specs=pl.BlockSpec((br, d), lambda i: (i, 0)),
        out_shape=jax.ShapeDtypeStruct((s, d), jnp.float32),
    )(p, cnt2)


def _final_body(g_ref, c_ref, w_ref, b_ref, o_ref):
    cnt = c_ref[...]
    deg = jnp.maximum(cnt, 1.0)
    z = (jnp.dot(g_ref[...], w_ref[...], preferred_element_type=jnp.float32)
         + cnt * b_ref[...]) / deg
    m = jnp.max(z, axis=-1, keepdims=True)
    e = jnp.exp(z - m)
    o_ref[...] = e / jnp.sum(e, axis=-1, keepdims=True)


def _tc_final(g, cnt2, W, b, br):
    s, d_in = g.shape
    d_out = W.shape[1]
    return pl.pallas_call(
        _final_body,
        grid=(s // br,),
        in_specs=[
            pl.BlockSpec((br, d_in), lambda i: (i, 0)),
            pl.BlockSpec((br, 1), lambda i: (i, 0)),
            pl.BlockSpec((d_in, d_out), lambda i: (0, 0)),
            pl.BlockSpec((1, d_out), lambda i: (0, 0)),
        ],
        out_specs=pl.BlockSpec((br, d_out), lambda i: (i, 0)),
        out_shape=jax.ShapeDtypeStruct((s, d_out), jnp.float32),
    )(g, cnt2, W, b)


def kernel(x, node_idx, hedge_idx, W1, b1, W2, b2):
    nidx = node_idx.astype(jnp.int32)
    hidx = hedge_idx.astype(jnp.int32)
    zrow = jnp.zeros((ZR, D1), jnp.float32)
    z1 = jnp.zeros((ZR,), jnp.float32)
    ones = jnp.ones((C,), jnp.float32)

    xp = jnp.concatenate([x, jnp.zeros((NVp - NV, D1), jnp.float32)], axis=0)
    xt1 = _tc_linear(xp, W1, b1.reshape(1, -1), br=1256)

    pA, cntE = _SWEEP_FULL(xt1, nidx, hidx, zrow, z1, ones)
    cntE3 = cntE.reshape(NC, NEp, 1)

    e1 = _tc_combine(pA, cntE3, br=1256)
    hB, cntV = _SWEEP_HALF(e1, hidx, nidx, zrow, z1, ones)
    cntV2 = cntV.reshape(NVp, 1)
    h = _tc_relu_norm(hB, cntV2, br=1256)
    pC, _ = _SWEEP_FULL(h, nidx, hidx, zrow, z1, ones)
    f1 = _tc_combine(pC, cntE3, br=1256)
    G, _ = _SWEEP_HALF(f1, hidx, nidx, zrow, z1, ones)
    out = _tc_final(G, cntV2, W2, b2.reshape(1, -1), br=1256)
    return out[:NV]


# speedup vs baseline: 8.4969x; 1.2185x over previous
import jax
import jax.numpy as jnp
from jax import lax
from jax.experimental import pallas as pl
from jax.experimental.pallas import tpu as pltpu
from jax.experimental.pallas import tpu_sc as plsc

NV = 10000
NE = 5000
NP = 320000
D1 = 128
D2 = 64

NC = 2
NS = 16
NW = NC * NS

NVp = 10048
NEp = 5024
SH = NVp // NC
DUMP = SH

C = 80
PW = NP // NW
NCHUNK = PW // C
CT_F = PW - NCHUNK * C
PWN = NP // NS
NCHUNK_N = PWN // C
CT_H = PWN - NCHUNK_N * C

_mesh = plsc.VectorSubcoreMesh(core_axis_name="c", subcore_axis_name="s")


RING = 4
ZR = 32
NZ = NEp // ZR
NZT = (NZ + NS - 1) // NS


def _run_pipelined(table, gidx_hbm, sidx_hbm, base, npairs, nchunks, ct,
                   remap,
                   idxg_f, idxs_f, idxgc, idxrc, idxgt, idxrt, rows, rowst,
                   ones_v, acc, dacc, gsem, scsem, sosem):
    pltpu.sync_copy(gidx_hbm.at[pl.ds(base, npairs)], idxg_f)
    pltpu.sync_copy(sidx_hbm.at[pl.ds(base, npairs)], idxs_f)

    def move_g(c, s):
        for j in range(C // 16):
            idxgc[s, pl.ds(j * 16, 16)] = idxg_f[pl.ds(c * C + j * 16, 16)]

    def move_s(c, s):
        for j in range(C // 16):
            idxrc[s, pl.ds(j * 16, 16)] = remap(
                idxs_f[pl.ds(c * C + j * 16, 16)])

    def issue_gather(c, s):
        move_g(c, s)
        pltpu.async_copy(table.at[idxgc.at[s]], rows.at[s], gsem.at[s])

    def wait_g(s):
        pltpu.make_async_copy(table.at[idxgc.at[s]], rows.at[s],
                              gsem.at[s]).wait()

    def issue_scatters(c, s):
        move_s(c, s)
        pltpu.async_copy(rows.at[s], acc.at[idxrc.at[s]], scsem.at[s],
                         add=True)
        pltpu.async_copy(ones_v, dacc.at[idxrc.at[s]], sosem.at[s],
                         add=True)

    def wait_sc(s):
        pltpu.make_async_copy(rows.at[s], acc.at[idxrc.at[s]],
                              scsem.at[s]).wait()
        pltpu.make_async_copy(ones_v, dacc.at[idxrc.at[s]],
                              sosem.at[s]).wait()

    for s in range(RING):
        issue_gather(s, s)

    K = (nchunks - RING) // RING
    E = nchunks - RING * K - RING

    @pl.loop(0, K)
    def _grp(k):
        for s in range(RING):
            wait_g(s)
            issue_scatters(RING * k + s, s)
        for s in range(RING):
            wait_sc(s)
            issue_gather(RING * k + RING + s, s)

    for s in range(RING):
        wait_g(s)
        issue_scatters(RING * K + s, s)
    for t in range(E):
        wait_sc(t)
        issue_gather(RING * K + RING + t, t)
        wait_g(t)
        issue_scatters(RING * K + RING + t, t)
    for s in range(RING):
        wait_sc(s)

    if not ct:
        return
    for j in range(ct // 16):
        idxgt[pl.ds(j * 16, 16)] = idxg_f[pl.ds(nchunks * C + j * 16, 16)]
        idxrt[pl.ds(j * 16, 16)] = remap(
            idxs_f[pl.ds(nchunks * C + j * 16, 16)])
    pltpu.async_copy(table.at[idxgt], rowst, gsem.at[0]).wait()
    pltpu.async_copy(rowst, acc.at[idxrt], scsem.at[0], add=True).wait()
    pltpu.async_copy(ones_v.at[pl.ds(0, ct)], dacc.at[idxrt],
                     sosem.at[0], add=True).wait()


def _sweep_full_body(table, gidx, sidx, zrow, z1, ones, pE, cntE,
                     idxg_f, idxs_f, idxgc, idxrc, idxgt, idxrt, rows, rowst,
                     zbuf, z1v, ones_v, accE, dE, gsem, scsem, sosem):
    cid = lax.axis_index("c")
    sid = lax.axis_index("s")
    wid = sid * NC + cid

    pltpu.sync_copy(zrow, zbuf)
    pltpu.sync_copy(z1, z1v)
    pltpu.sync_copy(ones, ones_v)
    for t in range(NZT):
        zi = sid + t * NS

        @pl.when(zi < NZ)
        def _zero():
            pltpu.sync_copy(zbuf, accE.at[pl.ds(zi * ZR, ZR)])
            pltpu.sync_copy(z1v, dE.at[pl.ds(zi * ZR, ZR)])

    plsc.subcore_barrier()

    _run_pipelined(table, gidx, sidx, wid * PW, PW, NCHUNK, CT_F,
                   lambda v: v,
                   idxg_f, idxs_f, idxgc, idxrc, idxgt, idxrt, rows, rowst,
                   ones_v, accE, dE, gsem, scsem, sosem)

    plsc.subcore_barrier()
    for t in range(NZT):
        zi = sid + t * NS

        @pl.when(zi < NZ)
        def _wout():
            pltpu.sync_copy(accE.at[pl.ds(zi * ZR, ZR)],
                            pE.at[cid, pl.ds(zi * ZR, ZR)])
            pltpu.sync_copy(dE.at[pl.ds(zi * ZR, ZR)], z1v)
            pltpu.sync_copy(z1v, cntE.at[pl.ds(cid * NEp + zi * ZR, ZR)])


_SWEEP_FULL = pl.kernel(
    _sweep_full_body,
    out_type=(
        jax.ShapeDtypeStruct((NC, NEp, D1), jnp.float32),
        jax.ShapeDtypeStruct((NC * NEp,), jnp.float32),
    ),
    mesh=_mesh,
    scratch_types=[
        pltpu.VMEM((PW,), jnp.int32),
        pltpu.VMEM((PW,), jnp.int32),
        pltpu.VMEM((RING, C), jnp.int32),
        pltpu.VMEM((RING, C), jnp.int32),
        pltpu.VMEM((max(CT_F,8),), jnp.int32),
        pltpu.VMEM((max(CT_F,8),), jnp.int32),
        pltpu.VMEM((RING, C, D1), jnp.float32),
        pltpu.VMEM((max(CT_F,8), D1), jnp.float32),
        pltpu.VMEM((ZR, D1), jnp.float32),
        pltpu.VMEM((ZR,), jnp.float32),
        pltpu.VMEM((C,), jnp.float32),
        pltpu.VMEM_SHARED((NEp, D1), jnp.float32),
        pltpu.VMEM_SHARED((NEp,), jnp.float32),
        pltpu.SemaphoreType.DMA((RING,)),
        pltpu.SemaphoreType.DMA((RING,)),
        pltpu.SemaphoreType.DMA((RING,)),
    ],
)


def _sweep_half_body(table, gidx, sidx, zrow, z1, ones, out, cntV,
                     idxg_f, idxs_f, idxgc, idxrc, idxgt, idxrt, rows, rowst,
                     zbuf, z1v, ones_v, acc, dV,
                     gsem, scsem, sosem):
    cid = lax.axis_index("c")
    sid = lax.axis_index("s")
    lo = cid * SH
    hi = lo + SH

    pltpu.sync_copy(zrow, zbuf)
    pltpu.sync_copy(z1, z1v)
    pltpu.sync_copy(ones, ones_v)
    for t in range(NZT):
        zi = sid + t * NS

        @pl.when(zi < NZ)
        def _zero():
            pltpu.sync_copy(zbuf, acc.at[pl.ds(zi * ZR, ZR)])
            pltpu.sync_copy(z1v, dV.at[pl.ds(zi * ZR, ZR)])

    plsc.subcore_barrier()

    def remap(sv):
        m = jnp.logical_and(sv >= lo, sv < hi)
        return jnp.where(m, sv - lo, DUMP + (sv & 7))

    _run_pipelined(table, gidx, sidx, sid * PWN, PWN, NCHUNK_N, CT_H,
                   remap,
                   idxg_f, idxs_f, idxgc, idxrc, idxgt, idxrt, rows, rowst,
                   ones_v, acc, dV, gsem, scsem, sosem)

    plsc.subcore_barrier()
    for t in range(NZT):
        zi = sid + t * NS

        @pl.when(zi < NZ)
        def _wout():
            pltpu.sync_copy(acc.at[pl.ds(zi * ZR, ZR)],
                            out.at[pl.ds(cid * SH + zi * ZR, ZR)])
            pltpu.sync_copy(dV.at[pl.ds(zi * ZR, ZR)], z1v)
            pltpu.sync_copy(z1v, cntV.at[pl.ds(cid * SH + zi * ZR, ZR)])


_SWEEP_HALF = pl.kernel(
    _sweep_half_body,
    out_type=(
        jax.ShapeDtypeStruct((NVp, D1), jnp.float32),
        jax.ShapeDtypeStruct((NVp,), jnp.float32),
    ),
    mesh=_mesh,
    scratch_types=[
        pltpu.VMEM((PWN,), jnp.int32),
        pltpu.VMEM((PWN,), jnp.int32),
        pltpu.VMEM((RING, C), jnp.int32),
        pltpu.VMEM((RING, C), jnp.int32),
        pltpu.VMEM((max(CT_H,8),), jnp.int32),
        pltpu.VMEM((max(CT_H,8),), jnp.int32),
        pltpu.VMEM((RING, C, D1), jnp.float32),
        pltpu.VMEM((max(CT_H,8), D1), jnp.float32),
        pltpu.VMEM((ZR, D1), jnp.float32),
        pltpu.VMEM((ZR,), jnp.float32),
        pltpu.VMEM((C,), jnp.float32),
        pltpu.VMEM_SHARED((SH + 8, D1), jnp.float32),
        pltpu.VMEM_SHARED((SH + 8,), jnp.float32),
        pltpu.SemaphoreType.DMA((RING,)),
        pltpu.SemaphoreType.DMA((RING,)),
        pltpu.SemaphoreType.DMA((RING,)),
    ],
)


def _linear_body(x_ref, w_ref, b_ref, o_ref):
    o_ref[...] = (
        jnp.dot(x_ref[...], w_ref[...], preferred_element_type=jnp.float32)
        + b_ref[...]
    )


def _tc_linear(x, W, b, br):
    n, d_in = x.shape
    d_out = W.shape[1]
    return pl.pallas_call(
        _linear_body,
        grid=(n // br,),
        in_specs=[
            pl.BlockSpec((br, d_in), lambda i: (i, 0)),
            pl.BlockSpec((d_in, d_out), lambda i: (0, 0)),
            pl.BlockSpec((1, d_out), lambda i: (0, 0)),
        ],
        out_specs=pl.BlockSpec((br, d_out), lambda i: (i, 0)),
        out_shape=jax.ShapeDtypeStruct((n, d_out), jnp.float32),
    )(x, W, b)


def _combine_body(p_ref, c_ref, o_ref):
    p = p_ref[...]
    deg = jnp.maximum(c_ref[0] + c_ref[1], 1.0)
    o_ref[...] = (p[0] + p[1]) / deg


def _tc_combine(p, cnt3, br):
    _, s, d = p.shape
    return pl.pallas_call(
        _combine_body,
        grid=(s // br,),
        in_specs=[
            pl.BlockSpec((NC, br, d), lambda i: (0, i, 0)),
            pl.BlockSpec((NC, br, 1), lambda i: (0, i, 0)),
        ],
        out_specs=pl.BlockSpec((br, d), lambda i: (i, 0)),
        out_shape=jax.ShapeDtypeStruct((s, d), jnp.float32),
    )(p, cnt3)


def _relu_norm_body(p_ref, c_ref, o_ref):
    deg = jnp.maximum(c_ref[...], 1.0)
    o_ref[...] = jnp.maximum(p_ref[...] / deg, 0.0)


def _tc_relu_norm(p, cnt2, br):
    s, d = p.shape
    return pl.pallas_call(
        _relu_norm_body,
        grid=(s // br,),
        in_specs=[
            pl.BlockSpec((br, d), lambda i: (i, 0)),
            pl.BlockSpec((br, 1), lambda i: (i, 0)),
        ],
        out_specs=pl.BlockSpec((br, d), lambda i: (i, 0)),
        out_shape=jax.ShapeDtypeStruct((s, d), jnp.float32),
    )(p, cnt2)


def _final_body(g_ref, c_ref, w_ref, b_ref, o_ref):
    cnt = c_ref[...]
    deg = jnp.maximum(cnt, 1.0)
    z = (jnp.dot(g_ref[...], w_ref[...], preferred_element_type=jnp.float32)
         + cnt * b_ref[...]) / deg
    m = jnp.max(z, axis=-1, keepdims=True)
    e = jnp.exp(z - m)
    o_ref[...] = e / jnp.sum(e, axis=-1, keepdims=True)


def _tc_final(g, cnt2, W, b, br):
    s, d_in = g.shape
    d_out = W.shape[1]
    return pl.pallas_call(
        _final_body,
        grid=(s // br,),
        in_specs=[
            pl.BlockSpec((br, d_in), lambda i: (i, 0)),
            pl.BlockSpec((br, 1), lambda i: (i, 0)),
            pl.BlockSpec((d_in, d_out), lambda i: (0, 0)),
            pl.BlockSpec((1, d_out), lambda i: (0, 0)),
        ],
        out_specs=pl.BlockSpec((br, d_out), lambda i: (i, 0)),
        out_shape=jax.ShapeDtypeStruct((s, d_out), jnp.float32),
    )(g, cnt2, W, b)


def kernel(x, node_idx, hedge_idx, W1, b1, W2, b2):
    nidx = node_idx.astype(jnp.int32)
    hidx = hedge_idx.astype(jnp.int32)
    zrow = jnp.zeros((ZR, D1), jnp.float32)
    z1 = jnp.zeros((ZR,), jnp.float32)
    ones = jnp.ones((C,), jnp.float32)

    xp = jnp.concatenate([x, jnp.zeros((NVp - NV, D1), jnp.float32)], axis=0)
    xt1 = _tc_linear(xp, W1, b1.reshape(1, -1), br=1256)

    pA, cntE = _SWEEP_FULL(xt1, nidx, hidx, zrow, z1, ones)
    cntE3 = cntE.reshape(NC, NEp, 1)

    e1 = _tc_combine(pA, cntE3, br=1256)
    hB, cntV = _SWEEP_HALF(e1, hidx, nidx, zrow, z1, ones)
    cntV2 = cntV.reshape(NVp, 1)
    h = _tc_relu_norm(hB, cntV2, br=1256)
    pC, _ = _SWEEP_FULL(h, nidx, hidx, zrow, z1, ones)
    f1 = _tc_combine(pC, cntE3, br=1256)
    G, _ = _SWEEP_HALF(f1, hidx, nidx, zrow, z1, ones)
    out = _tc_final(G, cntV2, W2, b2.reshape(1, -1), br=1256)
    return out[:NV]


# speedup vs baseline: 8.5388x; 1.0049x over previous
import jax
import jax.numpy as jnp
from jax import lax
from jax.experimental import pallas as pl
from jax.experimental.pallas import tpu as pltpu
from jax.experimental.pallas import tpu_sc as plsc

NV = 10000
NE = 5000
NP = 320000
D1 = 128
D2 = 64

NC = 2
NS = 16
NW = NC * NS

NVp = 10048
NEp = 5024
SH = NVp // NC
DUMP = SH

C = 80
PW = NP // NW
NCHUNK = PW // C
CT_F = PW - NCHUNK * C
PWN = NP // NS
NCHUNK_N = PWN // C
CT_H = PWN - NCHUNK_N * C

_mesh = plsc.VectorSubcoreMesh(core_axis_name="c", subcore_axis_name="s")


RING = 4
ZR = 32
NZ = NEp // ZR
NZT = (NZ + NS - 1) // NS


def _run_pipelined(table, gidx_hbm, sidx_hbm, base, npairs, nchunks, ct,
                   remap, do_deg,
                   idxg_f, idxs_f, idxgc, idxrc, idxgt, idxrt, rows, rowst,
                   ones_v, acc, dacc, gsem, scsem, sosem):
    pltpu.sync_copy(gidx_hbm.at[pl.ds(base, npairs)], idxg_f)
    pltpu.sync_copy(sidx_hbm.at[pl.ds(base, npairs)], idxs_f)

    def move_g(c, s):
        for j in range(C // 16):
            idxgc[s, pl.ds(j * 16, 16)] = idxg_f[pl.ds(c * C + j * 16, 16)]

    def move_s(c, s):
        for j in range(C // 16):
            idxrc[s, pl.ds(j * 16, 16)] = remap(
                idxs_f[pl.ds(c * C + j * 16, 16)])

    def issue_gather(c, s):
        move_g(c, s)
        pltpu.async_copy(table.at[idxgc.at[s]], rows.at[s], gsem.at[s])

    def wait_g(s):
        pltpu.make_async_copy(table.at[idxgc.at[s]], rows.at[s],
                              gsem.at[s]).wait()

    def issue_scatters(c, s):
        move_s(c, s)
        pltpu.async_copy(rows.at[s], acc.at[idxrc.at[s]], scsem.at[s],
                         add=True)

        @pl.when(do_deg)
        def _deg():
            pltpu.async_copy(ones_v, dacc.at[idxrc.at[s]], sosem.at[s],
                             add=True)

    def wait_sc(s):
        pltpu.make_async_copy(rows.at[s], acc.at[idxrc.at[s]],
                              scsem.at[s]).wait()

        @pl.when(do_deg)
        def _degw():
            pltpu.make_async_copy(ones_v, dacc.at[idxrc.at[s]],
                                  sosem.at[s]).wait()

    for s in range(RING):
        issue_gather(s, s)

    K = (nchunks - RING) // RING
    E = nchunks - RING * K - RING

    @pl.loop(0, K)
    def _grp(k):
        for s in range(RING):
            wait_g(s)
            issue_scatters(RING * k + s, s)
        for s in range(RING):
            wait_sc(s)
            issue_gather(RING * k + RING + s, s)

    for s in range(RING):
        wait_g(s)
        issue_scatters(RING * K + s, s)
    for t in range(E):
        wait_sc(t)
        issue_gather(RING * K + RING + t, t)
        wait_g(t)
        issue_scatters(RING * K + RING + t, t)
    for s in range(RING):
        wait_sc(s)

    if not ct:
        return
    for j in range(ct // 16):
        idxgt[pl.ds(j * 16, 16)] = idxg_f[pl.ds(nchunks * C + j * 16, 16)]
        idxrt[pl.ds(j * 16, 16)] = remap(
            idxs_f[pl.ds(nchunks * C + j * 16, 16)])
    pltpu.async_copy(table.at[idxgt], rowst, gsem.at[0]).wait()
    pltpu.async_copy(rowst, acc.at[idxrt], scsem.at[0], add=True).wait()

    @pl.when(do_deg)
    def _degt():
        pltpu.async_copy(ones_v.at[pl.ds(0, ct)], dacc.at[idxrt],
                         sosem.at[0], add=True).wait()


def _sweep_full_body(table, gidx, sidx, zrow, z1, ones, flag, pE, cntE,
                     idxg_f, idxs_f, idxgc, idxrc, idxgt, idxrt, rows, rowst,
                     zbuf, z1v, ones_v, flagv, accE, dE, gsem, scsem, sosem):
    cid = lax.axis_index("c")
    sid = lax.axis_index("s")
    wid = sid * NC + cid

    pltpu.sync_copy(zrow, zbuf)
    pltpu.sync_copy(z1, z1v)
    pltpu.sync_copy(ones, ones_v)
    for t in range(NZT):
        zi = sid + t * NS

        @pl.when(zi < NZ)
        def _zero():
            pltpu.sync_copy(zbuf, accE.at[pl.ds(zi * ZR, ZR)])
            pltpu.sync_copy(z1v, dE.at[pl.ds(zi * ZR, ZR)])

    plsc.subcore_barrier()

    pltpu.sync_copy(flag, flagv)
    _run_pipelined(table, gidx, sidx, wid * PW, PW, NCHUNK, CT_F,
                   lambda v: v, flagv[...][0] > 0,
                   idxg_f, idxs_f, idxgc, idxrc, idxgt, idxrt, rows, rowst,
                   ones_v, accE, dE, gsem, scsem, sosem)

    plsc.subcore_barrier()
    for t in range(NZT):
        zi = sid + t * NS

        @pl.when(zi < NZ)
        def _wout():
            pltpu.sync_copy(accE.at[pl.ds(zi * ZR, ZR)],
                            pE.at[cid, pl.ds(zi * ZR, ZR)])
            pltpu.sync_copy(dE.at[pl.ds(zi * ZR, ZR)], z1v)
            pltpu.sync_copy(z1v, cntE.at[pl.ds(cid * NEp + zi * ZR, ZR)])


_SWEEP_FULL = pl.kernel(
    _sweep_full_body,
    out_type=(
        jax.ShapeDtypeStruct((NC, NEp, D1), jnp.float32),
        jax.ShapeDtypeStruct((NC * NEp,), jnp.float32),
    ),
    mesh=_mesh,
    scratch_types=[
        pltpu.VMEM((PW,), jnp.int32),
        pltpu.VMEM((PW,), jnp.int32),
        pltpu.VMEM((RING, C), jnp.int32),
        pltpu.VMEM((RING, C), jnp.int32),
        pltpu.VMEM((max(CT_F,8),), jnp.int32),
        pltpu.VMEM((max(CT_F,8),), jnp.int32),
        pltpu.VMEM((RING, C, D1), jnp.float32),
        pltpu.VMEM((max(CT_F,8), D1), jnp.float32),
        pltpu.VMEM((ZR, D1), jnp.float32),
        pltpu.VMEM((ZR,), jnp.float32),
        pltpu.VMEM((C,), jnp.float32),
        pltpu.VMEM((16,), jnp.int32),
        pltpu.VMEM_SHARED((NEp, D1), jnp.float32),
        pltpu.VMEM_SHARED((NEp,), jnp.float32),
        pltpu.SemaphoreType.DMA((RING,)),
        pltpu.SemaphoreType.DMA((RING,)),
        pltpu.SemaphoreType.DMA((RING,)),
    ],
)


def _sweep_half_body(table, gidx, sidx, zrow, z1, ones, flag, out, cntV,
                     idxg_f, idxs_f, idxgc, idxrc, idxgt, idxrt, rows, rowst,
                     zbuf, z1v, ones_v, flagv, acc, dV,
                     gsem, scsem, sosem):
    cid = lax.axis_index("c")
    sid = lax.axis_index("s")
    lo = cid * SH
    hi = lo + SH

    pltpu.sync_copy(zrow, zbuf)
    pltpu.sync_copy(z1, z1v)
    pltpu.sync_copy(ones, ones_v)
    for t in range(NZT):
        zi = sid + t * NS

        @pl.when(zi < NZ)
        def _zero():
            pltpu.sync_copy(zbuf, acc.at[pl.ds(zi * ZR, ZR)])
            pltpu.sync_copy(z1v, dV.at[pl.ds(zi * ZR, ZR)])

    plsc.subcore_barrier()

    def remap(sv):
        m = jnp.logical_and(sv >= lo, sv < hi)
        return jnp.where(m, sv - lo, DUMP + (sv & 7))

    pltpu.sync_copy(flag, flagv)
    _run_pipelined(table, gidx, sidx, sid * PWN, PWN, NCHUNK_N, CT_H,
                   remap, flagv[...][0] > 0,
                   idxg_f, idxs_f, idxgc, idxrc, idxgt, idxrt, rows, rowst,
                   ones_v, acc, dV, gsem, scsem, sosem)

    plsc.subcore_barrier()
    for t in range(NZT):
        zi = sid + t * NS

        @pl.when(zi < NZ)
        def _wout():
            pltpu.sync_copy(acc.at[pl.ds(zi * ZR, ZR)],
                            out.at[pl.ds(cid * SH + zi * ZR, ZR)])
            pltpu.sync_copy(dV.at[pl.ds(zi * ZR, ZR)], z1v)
            pltpu.sync_copy(z1v, cntV.at[pl.ds(cid * SH + zi * ZR, ZR)])


_SWEEP_HALF = pl.kernel(
    _sweep_half_body,
    out_type=(
        jax.ShapeDtypeStruct((NVp, D1), jnp.float32),
        jax.ShapeDtypeStruct((NVp,), jnp.float32),
    ),
    mesh=_mesh,
    scratch_types=[
        pltpu.VMEM((PWN,), jnp.int32),
        pltpu.VMEM((PWN,), jnp.int32),
        pltpu.VMEM((RING, C), jnp.int32),
        pltpu.VMEM((RING, C), jnp.int32),
        pltpu.VMEM((max(CT_H,8),), jnp.int32),
        pltpu.VMEM((max(CT_H,8),), jnp.int32),
        pltpu.VMEM((RING, C, D1), jnp.float32),
        pltpu.VMEM((max(CT_H,8), D1), jnp.float32),
        pltpu.VMEM((ZR, D1), jnp.float32),
        pltpu.VMEM((ZR,), jnp.float32),
        pltpu.VMEM((C,), jnp.float32),
        pltpu.VMEM((16,), jnp.int32),
        pltpu.VMEM_SHARED((SH + 8, D1), jnp.float32),
        pltpu.VMEM_SHARED((SH + 8,), jnp.float32),
        pltpu.SemaphoreType.DMA((RING,)),
        pltpu.SemaphoreType.DMA((RING,)),
        pltpu.SemaphoreType.DMA((RING,)),
    ],
)


def _linear_body(x_ref, w_ref, b_ref, o_ref):
    o_ref[...] = (
        jnp.dot(x_ref[...], w_ref[...], preferred_element_type=jnp.float32)
        + b_ref[...]
    )


def _tc_linear(x, W, b, br):
    n, d_in = x.shape
    d_out = W.shape[1]
    return pl.pallas_call(
        _linear_body,
        grid=(n // br,),
        in_specs=[
            pl.BlockSpec((br, d_in), lambda i: (i, 0)),
            pl.BlockSpec((d_in, d_out), lambda i: (0, 0)),
            pl.BlockSpec((1, d_out), lambda i: (0, 0)),
        ],
        out_specs=pl.BlockSpec((br, d_out), lambda i: (i, 0)),
        out_shape=jax.ShapeDtypeStruct((n, d_out), jnp.float32),
    )(x, W, b)


def _combine_body(p_ref, c_ref, o_ref):
    p = p_ref[...]
    deg = jnp.maximum(c_ref[0] + c_ref[1], 1.0)
    o_ref[...] = (p[0] + p[1]) / deg


def _tc_combine(p, cnt3, br):
    _, s, d = p.shape
    return pl.pallas_call(
        _combine_body,
        grid=(s // br,),
        in_specs=[
            pl.BlockSpec((NC, br, d), lambda i: (0, i, 0)),
            pl.BlockSpec((NC, br, 1), lambda i: (0, i, 0)),
        ],
        out_specs=pl.BlockSpec((br, d), lambda i: (i, 0)),
        out_shape=jax.ShapeDtypeStruct((s, d), jnp.float32),
    )(p, cnt3)


def _relu_norm_body(p_ref, c_ref, o_ref):
    deg = jnp.maximum(c_ref[...], 1.0)
    o_ref[...] = jnp.maximum(p_ref[...] / deg, 0.0)


def _tc_relu_norm(p, cnt2, br):
    s, d = p.shape
    return pl.pallas_call(
        _relu_norm_body,
        grid=(s // br,),
        in_specs=[
            pl.BlockSpec((br, d), lambda i: (i, 0)),
            pl.BlockSpec((br, 1), lambda i: (i, 0)),
        ],
        out_specs=pl.BlockSpec((br, d), lambda i: (i, 0)),
        out_shape=jax.ShapeDtypeStruct((s, d), jnp.float32),
    )(p, cnt2)


def _final_body(g_ref, c_ref, w_ref, b_ref, o_ref):
    cnt = c_ref[...]
    deg = jnp.maximum(cnt, 1.0)
    z = (jnp.dot(g_ref[...], w_ref[...], preferred_element_type=jnp.float32)
         + cnt * b_ref[...]) / deg
    m = jnp.max(z, axis=-1, keepdims=True)
    e = jnp.exp(z - m)
    o_ref[...] = e / jnp.sum(e, axis=-1, keepdims=True)


def _tc_final(g, cnt2, W, b, br):
    s, d_in = g.shape
    d_out = W.shape[1]
    return pl.pallas_call(
        _final_body,
        grid=(s // br,),
        in_specs=[
            pl.BlockSpec((br, d_in), lambda i: (i, 0)),
            pl.BlockSpec((br, 1), lambda i: (i, 0)),
            pl.BlockSpec((d_in, d_out), lambda i: (0, 0)),
            pl.BlockSpec((1, d_out), lambda i: (0, 0)),
        ],
        out_specs=pl.BlockSpec((br, d_out), lambda i: (i, 0)),
        out_shape=jax.ShapeDtypeStruct((s, d_out), jnp.float32),
    )(g, cnt2, W, b)


def kernel(x, node_idx, hedge_idx, W1, b1, W2, b2):
    nidx = node_idx.astype(jnp.int32)
    hidx = hedge_idx.astype(jnp.int32)
    zrow = jnp.zeros((ZR, D1), jnp.float32)
    z1 = jnp.zeros((ZR,), jnp.float32)
    ones = jnp.ones((C,), jnp.float32)
    deg_on = jnp.ones((16,), jnp.int32)
    deg_off = jnp.zeros((16,), jnp.int32)

    xp = jnp.concatenate([x, jnp.zeros((NVp - NV, D1), jnp.float32)], axis=0)
    xt1 = _tc_linear(xp, W1, b1.reshape(1, -1), br=1256)

    pA, cntE = _SWEEP_FULL(xt1, nidx, hidx, zrow, z1, ones, deg_on)
    cntE3 = cntE.reshape(NC, NEp, 1)

    e1 = _tc_combine(pA, cntE3, br=1256)
    hB, cntV = _SWEEP_HALF(e1, hidx, nidx, zrow, z1, ones, deg_on)
    cntV2 = cntV.reshape(NVp, 1)
    h = _tc_relu_norm(hB, cntV2, br=1256)
    pC, _ = _SWEEP_FULL(h, nidx, hidx, zrow, z1, ones, deg_off)
    f1 = _tc_combine(pC, cntE3, br=1256)
    G, _ = _SWEEP_HALF(f1, hidx, nidx, zrow, z1, ones, deg_off)
    out = _tc_final(G, cntV2, W2, b2.reshape(1, -1), br=1256)
    return out[:NV]
